# baseline (speedup 1.0000x reference)
import jax
import jax.numpy as jnp
from jax.experimental import pallas as pl
from jax.experimental.pallas import tpu as pltpu

_B, _T, _D = 4, 4096, 2048
_W = 128
_K = 1024
_TT = 256
_NT = _T // _TT


def _prefix_sum(x):
    n = x.shape[1]
    s = 1
    while s < n:
        z = jnp.zeros((x.shape[0], s), x.dtype)
        x = x + jnp.concatenate([z, x[:, : n - s]], axis=1)
        s *= 2
    return x


def _fused_body(scal_ref, a_ref, p_ref, g_ref, m_ref,
                carry_ref, tail_ref, bnd_ref):
    t = pl.program_id(0)
    c_ce = scal_ref[0]
    m_cu = scal_ref[1]
    bce = scal_ref[2]
    bcu = scal_ref[3]

    a = a_ref[...]
    p = p_ref[...]
    inv_d = jnp.float32(1.0 / _D)
    d = a - p
    d_st = jnp.sum(a * a, axis=-1) * inv_d
    d_ch = jnp.sum(d * d, axis=-1) * inv_d

    @pl.when(t == 0)
    def _():
        carry_ref[...] = jnp.zeros((_B, 1), jnp.float32)
        tail_ref[...] = jnp.zeros((_B, _W), jnp.float32)
        bnd_ref[...] = jnp.concatenate(
            [jnp.full((_B, 1), 0x7F000000, jnp.int32),
             jnp.zeros((_B, 1), jnp.int32)], axis=1)

    csum = _prefix_sum(d_st) + carry_ref[...]
    carry_ref[...] = csum[:, _TT - 1:]
    shifted = jnp.concatenate(
        [tail_ref[...], csum[:, : _TT - _W]], axis=1)
    tail_ref[...] = csum[:, _TT - _W:]

    pos = (jax.lax.broadcasted_iota(jnp.int32, (_B, _TT), 1)
           + t * _TT).astype(jnp.float32)
    counts = jnp.minimum(pos + 1.0, jnp.float32(_W))
    cu = d_st - m_cu * ((csum - shifted) / counts)
    ce = d_st - (d_ch - c_ce)

    s_ce = 1.0 / (1.0 + jnp.exp(-bce * ce))
    s_cu = 1.0 / (1.0 + jnp.exp(-bcu * cu))
    g = s_ce + s_cu - s_ce * s_cu
    g_ref[:, pl.ds(t * _TT, _TT)] = g

    gbits = jax.lax.bitcast_convert_type(g, jnp.int32)
    cmin = jnp.minimum(bnd_ref[:, 0:1], jnp.min(gbits, axis=1, keepdims=True))
    cmax = jnp.maximum(bnd_ref[:, 1:2], jnp.max(gbits, axis=1, keepdims=True))
    bnd_ref[...] = jnp.concatenate([cmin, cmax], axis=1)

    @pl.when(t == _NT - 1)
    def _():
        bits = jax.lax.bitcast_convert_type(g_ref[...], jnp.int32)
        lo = bnd_ref[:, 0:1] - 1
        hi = bnd_ref[:, 1:2] + 1

        def body(_, carry):
            lo, hi = carry
            mid = lo + (hi - lo) // 2
            cnt = jnp.sum((bits >= mid).astype(jnp.int32), axis=1,
                          keepdims=True)
            ge = cnt >= _K
            return jnp.where(ge, mid, lo), jnp.where(ge, hi, mid)

        def cond_iter(carry):
            lo, hi = carry
            return jnp.any(hi - lo > 1)

        lo, hi = jax.lax.while_loop(cond_iter, lambda c: body(0, c), (lo, hi))
        tau = lo
        gt = bits > tau
        eq = bits == tau
        cnt_gt = jnp.sum(gt.astype(jnp.int32), axis=1, keepdims=True)
        need = _K - cnt_gt
        eq_rank = _prefix_sum(eq.astype(jnp.int32))
        mask = gt | (eq & (eq_rank <= need))
        m_ref[...] = mask.astype(jnp.float32)


def kernel(actual_residual, predicted_residual, raw_o_ce, raw_m_cu, beta_ce, beta_cu):
    o_ce_pos = jax.nn.softplus(jnp.asarray(raw_o_ce, jnp.float32))
    m_cu_pos = jax.nn.softplus(jnp.asarray(raw_m_cu, jnp.float32))
    scal = jnp.stack([
        jnp.log(o_ce_pos + 1e-10),
        m_cu_pos,
        jnp.asarray(beta_ce, jnp.float32),
        jnp.asarray(beta_cu, jnp.float32),
    ])

    g, mask = pl.pallas_call(
        _fused_body,
        grid=(_NT,),
        in_specs=[
            pl.BlockSpec(memory_space=pltpu.SMEM),
            pl.BlockSpec((_B, _TT, _D), lambda t: (0, t, 0)),
            pl.BlockSpec((_B, _TT, _D), lambda t: (0, t, 0)),
        ],
        out_specs=[
            pl.BlockSpec((_B, _T), lambda t: (0, 0)),
            pl.BlockSpec((_B, _T), lambda t: (0, 0)),
        ],
        out_shape=[
            jax.ShapeDtypeStruct((_B, _T), jnp.float32),
            jax.ShapeDtypeStruct((_B, _T), jnp.float32),
        ],
        scratch_shapes=[
            pltpu.VMEM((_B, 1), jnp.float32),
            pltpu.VMEM((_B, _W), jnp.float32),
            pltpu.VMEM((_B, 2), jnp.int32),
        ],
    )(scal, actual_residual, predicted_residual)
    return (g, mask)
